# baseline (speedup 1.0000x reference)
import jax
import jax.numpy as jnp
from jax.experimental import pallas as pl
from jax.experimental.pallas import tpu as pltpu

_N = 1048576
_H = 64
_TAU = 1000.0
_TILE = 16384
_NT = _N // _TILE


def _argmin_gather_kernel(ts_ref, mb_hbm, min_ref, row_ref, sem):
    x = ts_ref[...]
    m = jnp.min(x)
    r, c = x.shape
    lin = (jax.lax.broadcasted_iota(jnp.int32, (r, c), 0) * c
           + jax.lax.broadcasted_iota(jnp.int32, (r, c), 1))
    cand = jnp.where(x == m, lin, jnp.int32(2147483647))
    idx = jnp.min(cand)
    min_ref[0] = m
    cp = pltpu.make_async_copy(mb_hbm.at[pl.ds(idx, 1)], row_ref, sem)
    cp.start()
    cp.wait()


def _stream_kernel(ts_ref, mb_ref, out_ref):
    i = pl.program_id(0)
    w = jnp.exp(ts_ref[0] * (1.0 / _TAU))
    part = jax.lax.dot_general(
        w, mb_ref[...],
        dimension_numbers=(((1,), (0,)), ((), ())),
        preferred_element_type=jnp.float32,
    )

    @pl.when(i == 0)
    def _init():
        out_ref[...] = part

    @pl.when(i != 0)
    def _acc():
        out_ref[...] = out_ref[...] + part


def kernel(query_embedding, embedding, timestamp, current_timestamp,
           memory_bank, timestamps):
    min_ts, row = pl.pallas_call(
        _argmin_gather_kernel,
        in_specs=[
            pl.BlockSpec(memory_space=pltpu.VMEM),
            pl.BlockSpec(memory_space=pl.ANY),
        ],
        out_specs=[
            pl.BlockSpec(memory_space=pltpu.SMEM),
            pl.BlockSpec(memory_space=pltpu.VMEM),
        ],
        out_shape=[
            jax.ShapeDtypeStruct((1,), jnp.float32),
            jax.ShapeDtypeStruct((1, _H), jnp.float32),
        ],
        scratch_shapes=[pltpu.SemaphoreType.DMA],
    )(timestamps.reshape(_N // 128, 128), memory_bank)

    s = pl.pallas_call(
        _stream_kernel,
        grid=(_NT,),
        in_specs=[
            pl.BlockSpec((1, 1, _TILE), lambda i: (i, 0, 0)),
            pl.BlockSpec((_TILE, _H), lambda i: (i, 0)),
        ],
        out_specs=pl.BlockSpec((1, _H), lambda i: (0, 0)),
        out_shape=jax.ShapeDtypeStruct((1, _H), jnp.float32),
    )(timestamps.reshape(_NT, 1, _TILE), memory_bank)

    scale = jnp.exp(-current_timestamp / _TAU)
    w_old = jnp.exp((min_ts[0] - current_timestamp) / _TAU)
    w_new = jnp.exp((timestamp - current_timestamp) / _TAU)
    out = (s[0] * scale - w_old * row[0] + w_new * embedding) * (1.0 / _N)
    return out
